# baseline (speedup 1.0000x reference)
import functools

import jax
import jax.numpy as jnp
from jax import lax
from jax.experimental import pallas as pl
from jax.experimental.pallas import tpu as pltpu
from jax.experimental.pallas import tpu_sc as plsc

_VOCAB = 1000001
_DIM = 96
_BATCH = 16384

_INFO = plsc.get_sparse_core_info()
_NC = _INFO.num_cores
_NS = _INFO.num_subcores
_NW = _NC * _NS

_SPLIT = 10240
_B_PER_W = _SPLIT // _NW


@functools.partial(
    pl.kernel,
    mesh=plsc.VectorSubcoreMesh(core_axis_name="c", subcore_axis_name="s"),
    out_type=jax.ShapeDtypeStruct((_SPLIT, _DIM), jnp.float32),
    scratch_types=[
        pltpu.VMEM((_B_PER_W,), jnp.int32),
        pltpu.VMEM((_B_PER_W, _DIM), jnp.float32),
        pltpu.SemaphoreType.DMA,
    ],
    cost_estimate=pl.CostEstimate(
        flops=0, transcendentals=0, bytes_accessed=400_000_000
    ),
)
def _sc_gather(idx_hbm, table_hbm, out_hbm, idx_v, rows_v, sem):
    wid = lax.axis_index("s") * _NC + lax.axis_index("c")
    base = wid * _B_PER_W
    pltpu.sync_copy(idx_hbm.at[pl.ds(base, _B_PER_W)], idx_v)

    def body(blk):
        vec = idx_v[pl.ds(blk * 16, 16)]
        for l in range(16):
            i = vec[l]
            pltpu.make_async_copy(
                table_hbm.at[i], rows_v.at[blk * 16 + l], sem
            ).start()

    pl.loop(0, _B_PER_W // 16)(body)
    pltpu.make_async_copy(out_hbm.at[pl.ds(0, _B_PER_W)], rows_v, sem).wait()
    pltpu.sync_copy(rows_v, out_hbm.at[pl.ds(base, _B_PER_W)])


def _tc_gather_body(idx_smem, table_hbm, out_hbm, sem):
    def body(b, carry):
        i = idx_smem[_SPLIT + b]
        pltpu.make_async_copy(table_hbm.at[i], out_hbm.at[b], sem).start()
        return carry

    lax.fori_loop(0, _BATCH - _SPLIT, body, 0)
    pltpu.make_async_copy(out_hbm, out_hbm, sem).wait()


def _tc_gather(broadcaster, table):
    grid_spec = pltpu.PrefetchScalarGridSpec(
        num_scalar_prefetch=1,
        grid=(1,),
        in_specs=[pl.BlockSpec(memory_space=pl.ANY)],
        out_specs=pl.BlockSpec(memory_space=pl.ANY),
        scratch_shapes=[pltpu.SemaphoreType.DMA],
    )
    return pl.pallas_call(
        _tc_gather_body,
        grid_spec=grid_spec,
        out_shape=jax.ShapeDtypeStruct((_BATCH - _SPLIT, _DIM), jnp.float32),
        cost_estimate=pl.CostEstimate(
            flops=0, transcendentals=0, bytes_accessed=400_000_000
        ),
    )(broadcaster, table)


def kernel(broadcaster, table):
    tc_out = _tc_gather(broadcaster, table)
    sc_out = _sc_gather(broadcaster, table)
    return jnp.concatenate([sc_out, tc_out], axis=0)


# speedup vs baseline: 1.2319x; 1.2319x over previous
import functools

import jax
import jax.numpy as jnp
from jax import lax
from jax.experimental import pallas as pl
from jax.experimental.pallas import tpu as pltpu
from jax.experimental.pallas import tpu_sc as plsc

_VOCAB = 1000001
_DIM = 96
_BATCH = 16384

_INFO = plsc.get_sparse_core_info()
_NC = _INFO.num_cores
_NS = _INFO.num_subcores
_NW = _NC * _NS
_B_PER_W = _BATCH // _NW
_NSEM = 4


@functools.partial(
    pl.kernel,
    mesh=plsc.VectorSubcoreMesh(core_axis_name="c", subcore_axis_name="s"),
    out_type=jax.ShapeDtypeStruct((_BATCH, _DIM), jnp.float32),
    scratch_types=[
        pltpu.VMEM((_B_PER_W,), jnp.int32),
        pltpu.VMEM((_B_PER_W, _DIM), jnp.float32),
        [pltpu.SemaphoreType.DMA] * _NSEM,
    ],
)
def _sc_gather(idx_hbm, table_hbm, out_hbm, idx_v, rows_v, sems):
    wid = lax.axis_index("s") * _NC + lax.axis_index("c")
    base = wid * _B_PER_W
    pltpu.sync_copy(idx_hbm.at[pl.ds(base, _B_PER_W)], idx_v)

    def body(blk):
        vec = idx_v[pl.ds(blk * 16, 16)]
        for l in range(16):
            i = vec[l]
            pltpu.make_async_copy(
                table_hbm.at[i], rows_v.at[blk * 16 + l], sems[l % _NSEM]
            ).start()

    pl.loop(0, _B_PER_W // 16)(body)
    for s in range(_NSEM):
        pltpu.make_async_copy(
            out_hbm.at[pl.ds(0, _B_PER_W // _NSEM)],
            rows_v.at[pl.ds(0, _B_PER_W // _NSEM)],
            sems[s],
        ).wait()
    pltpu.sync_copy(rows_v, out_hbm.at[pl.ds(base, _B_PER_W)])


def kernel(broadcaster, table):
    return _sc_gather(broadcaster, table)
